# baseline (speedup 1.0000x reference)
import functools

import jax
import jax.numpy as jnp
from jax.experimental import pallas as pl
from jax.experimental.pallas import tpu as pltpu

D = 128
BZ = 10
RATIO = 512
CHUNKS = 128
LANES = 1024
NEG = float(jnp.finfo(jnp.float32).min)


def _score_body(x_ref, w_ref, o_ref):
    w = w_ref[...]
    w_norm = jnp.sqrt(jnp.sum(w * w, axis=-1, keepdims=True))
    s = jnp.sum(x_ref[...] * w, axis=-1, keepdims=True)
    o_ref[...] = jnp.tanh(s / w_norm)


def _select_body(score_ref, batch_ref, perm_ref, ts_ref, s_ref, cm_ref):
    b = pl.program_id(0)
    s = jnp.where(batch_ref[...] == b, score_ref[...], NEG)
    s_ref[...] = s
    cm_ref[...] = jnp.max(s, axis=1, keepdims=True)

    def step(j, _):
        c = jnp.argmax(cm_ref[...]).astype(jnp.int32)
        row = s_ref[pl.ds(c, 1), :]
        jj = jnp.argmax(row).astype(jnp.int32)
        val = jnp.max(row)
        perm_ref[pl.ds(j, 1), :] = jnp.reshape(c * LANES + jj, (1, 1))
        ts_ref[pl.ds(j, 1), :] = jnp.reshape(val, (1, 1))
        lane = jax.lax.broadcasted_iota(jnp.int32, (1, LANES), 1)
        row2 = jnp.where(lane == jj, NEG, row)
        s_ref[pl.ds(c, 1), :] = row2
        cm_ref[pl.ds(c, 1), :] = jnp.reshape(jnp.max(row2), (1, 1))
        return 0

    jax.lax.fori_loop(0, RATIO, step, 0)


def _gather_body(perm_ref, x_ref, ts_ref, o_ref):
    o_ref[...] = x_ref[...] * ts_ref[0, 0, 0]


def kernel(x, batch, weight):
    n = x.shape[0]

    score = pl.pallas_call(
        _score_body,
        grid=(pl.cdiv(n, 8192),),
        in_specs=[
            pl.BlockSpec((8192, D), lambda i: (i, 0)),
            pl.BlockSpec((1, D), lambda i: (0, 0)),
        ],
        out_specs=pl.BlockSpec((8192, 1), lambda i: (i, 0)),
        out_shape=jax.ShapeDtypeStruct((n, 1), jnp.float32),
    )(x, weight)

    pad = CHUNKS * LANES - n
    score_p = jnp.pad(score[:, 0], (0, pad)).reshape(CHUNKS, LANES)
    batch_p = jnp.pad(batch.astype(jnp.int32), (0, pad),
                      constant_values=BZ).reshape(CHUNKS, LANES)

    perm2, ts2 = pl.pallas_call(
        _select_body,
        grid=(BZ,),
        in_specs=[
            pl.BlockSpec((CHUNKS, LANES), lambda b: (0, 0)),
            pl.BlockSpec((CHUNKS, LANES), lambda b: (0, 0)),
        ],
        out_specs=[
            pl.BlockSpec((RATIO, 1), lambda b: (b, 0)),
            pl.BlockSpec((RATIO, 1), lambda b: (b, 0)),
        ],
        out_shape=[
            jax.ShapeDtypeStruct((BZ * RATIO, 1), jnp.int32),
            jax.ShapeDtypeStruct((BZ * RATIO, 1), jnp.float32),
        ],
        scratch_shapes=[
            pltpu.VMEM((CHUNKS, LANES), jnp.float32),
            pltpu.VMEM((CHUNKS, 1), jnp.float32),
        ],
    )(score_p, batch_p)

    perm = perm2.reshape(-1)

    x_top = pl.pallas_call(
        _gather_body,
        grid_spec=pltpu.PrefetchScalarGridSpec(
            num_scalar_prefetch=1,
            grid=(BZ * RATIO,),
            in_specs=[
                pl.BlockSpec((1, 1, D), lambda i, p: (p[i], 0, 0)),
                pl.BlockSpec((1, 1, 1), lambda i, p: (i, 0, 0)),
            ],
            out_specs=pl.BlockSpec((1, 1, D), lambda i, p: (i, 0, 0)),
        ),
        out_shape=jax.ShapeDtypeStruct((BZ * RATIO, 1, D), jnp.float32),
    )(perm, x.reshape(n, 1, D), ts2.reshape(BZ * RATIO, 1, 1))

    return (x_top.reshape(BZ, RATIO, D), perm)
